# baseline (speedup 1.0000x reference)
import jax
import jax.numpy as jnp
from jax import lax
from jax.experimental import pallas as pl
from jax.experimental.pallas import tpu as pltpu

_B = 128
_S = 197
_LP = 20
_D = 768
_K = 4
_L = 2
_RS = 104
_WS = 112


def _read_halves(x_hbm, inbuf, sem_r, b, slot):
    return (
        pltpu.make_async_copy(
            x_hbm.at[b, pl.ds(0, _RS), :],
            inbuf.at[slot, pl.ds(0, _RS), :],
            sem_r.at[slot, 0],
        ),
        pltpu.make_async_copy(
            x_hbm.at[b, pl.ds(_RS, _S - _RS), :],
            inbuf.at[slot, pl.ds(_RS, _S - _RS), :],
            sem_r.at[slot, 1],
        ),
    )


def _write_halves(out_hbm, outbuf, sem_w, b, slot):
    return (
        pltpu.make_async_copy(
            outbuf.at[slot, pl.ds(0, _WS), :],
            out_hbm.at[b, pl.ds(0, _WS), :],
            sem_w.at[slot, 0],
        ),
        pltpu.make_async_copy(
            outbuf.at[slot, pl.ds(_WS, _LP + _S - _WS), :],
            out_hbm.at[b, pl.ds(_WS, _LP + _S - _WS), :],
            sem_w.at[slot, 1],
        ),
    )


def _body(idx_ref, p_ref, x_hbm, out_hbm, inbuf, outbuf, sem_r, sem_w):
    def step(t, carry):
        b_r = t
        slot_r = lax.rem(b_r, _K)

        @pl.when(b_r < _B)
        def _():
            pltpu.async_copy(
                x_hbm.at[b_r, pl.ds(0, _RS), :],
                inbuf.at[slot_r, pl.ds(0, _RS), :],
                sem_r.at[slot_r, 0], priority=0,
            )
            pltpu.async_copy(
                x_hbm.at[b_r, pl.ds(_RS, _S - _RS), :],
                inbuf.at[slot_r, pl.ds(_RS, _S - _RS), :],
                sem_r.at[slot_r, 1], priority=1,
            )

        b_w = t - _L
        slot_w = lax.rem(t + (_K - _L), _K)

        @pl.when(b_w >= 0)
        def _():
            for c in _read_halves(x_hbm, inbuf, sem_r, b_w, slot_w):
                c.wait()

            @pl.when(b_w >= _K)
            def _():
                for c in _write_halves(out_hbm, outbuf, sem_w, b_w - _K, slot_w):
                    c.wait()

            outbuf[slot_w, :_LP, :] = p_ref[idx_ref[b_w]]
            outbuf[slot_w, _LP:, :] = inbuf[slot_w]
            pltpu.async_copy(
                outbuf.at[slot_w, pl.ds(0, _WS), :],
                out_hbm.at[b_w, pl.ds(0, _WS), :],
                sem_w.at[slot_w, 0], priority=0,
            )
            pltpu.async_copy(
                outbuf.at[slot_w, pl.ds(_WS, _LP + _S - _WS), :],
                out_hbm.at[b_w, pl.ds(_WS, _LP + _S - _WS), :],
                sem_w.at[slot_w, 1], priority=1,
            )

        return carry

    lax.fori_loop(0, _B + _L, step, 0)

    for b in range(_B - _K, _B):
        for c in _write_halves(out_hbm, outbuf, sem_w, b, b % _K):
            c.wait()


def kernel(x, prompt_idx, prompts):
    idx = prompt_idx.astype(jnp.int32)
    out = pl.pallas_call(
        _body,
        out_shape=jax.ShapeDtypeStruct((_B, _LP + _S, _D), jnp.float32),
        in_specs=[
            pl.BlockSpec(memory_space=pltpu.MemorySpace.SMEM),
            pl.BlockSpec(memory_space=pltpu.MemorySpace.VMEM),
            pl.BlockSpec(memory_space=pl.ANY),
        ],
        out_specs=pl.BlockSpec(memory_space=pl.ANY),
        scratch_shapes=[
            pltpu.VMEM((_K, _S, _D), jnp.float32),
            pltpu.VMEM((_K, _LP + _S, _D), jnp.float32),
            pltpu.SemaphoreType.DMA((_K, 2)),
            pltpu.SemaphoreType.DMA((_K, 2)),
        ],
    )(idx, prompts, x)
    return out
